# Initial kernel scaffold; baseline (speedup 1.0000x reference)
import jax
import jax.numpy as jnp
from jax.experimental import pallas as pl


def kernel(s, a, w_all):
    raise NotImplementedError("write your pallas kernel here")


# speedup vs baseline: 1.8550x; 1.8550x over previous
import jax
import jax.numpy as jnp
from jax.experimental import pallas as pl
from jax.experimental.pallas import tpu as pltpu

_TB = 2048


def _make_body(s_dim, a_dim):
    row_bh = s_dim + a_dim
    row_wo = row_bh + 1
    row_bo = row_wo + 1

    def body(s_ref, a_ref, w_ref, q_ref):
        x = jnp.concatenate(
            [s_ref[...].astype(jnp.bfloat16), a_ref[...].astype(jnp.bfloat16)],
            axis=1)
        w = w_ref[:row_bh, :].astype(jnp.bfloat16)
        h = jnp.dot(x, w, preferred_element_type=jnp.float32)
        h = jnp.maximum(h + w_ref[row_bh:row_bh + 1, :], 0.0)
        q = jnp.sum(h * w_ref[row_wo:row_wo + 1, :], axis=-1, keepdims=True)
        q_ref[...] = q + w_ref[row_bo:row_bo + 1, 0:1]

    return body


def kernel(s, a, w_all):
    B, s_dim = s.shape
    a_dim = a.shape[1]
    k_rows, hidden = w_all.shape

    tb = min(_TB, B) if B % min(_TB, B) == 0 else _TB
    pad = (-B) % tb
    if pad:
        s = jnp.pad(s, ((0, pad), (0, 0)))
        a = jnp.pad(a, ((0, pad), (0, 0)))
    bp = B + pad

    q = pl.pallas_call(
        _make_body(s_dim, a_dim),
        out_shape=jax.ShapeDtypeStruct((bp, 1), jnp.float32),
        grid=(bp // tb,),
        in_specs=[
            pl.BlockSpec((tb, s_dim), lambda i: (i, 0)),
            pl.BlockSpec((tb, a_dim), lambda i: (i, 0)),
            pl.BlockSpec((k_rows, hidden), lambda i: (0, 0)),
        ],
        out_specs=pl.BlockSpec((tb, 1), lambda i: (i, 0)),
        compiler_params=pltpu.CompilerParams(
            dimension_semantics=("parallel",),
            vmem_limit_bytes=64 << 20,
        ),
    )(s, a, w_all)
    return q[:B]


# speedup vs baseline: 2.1812x; 1.1758x over previous
import jax
import jax.numpy as jnp
from jax.experimental import pallas as pl
from jax.experimental.pallas import tpu as pltpu

_TB = 4096


def _make_body(s_dim, a_dim):
    row_bh = s_dim + a_dim
    row_wo = row_bh + 1
    row_bo = row_wo + 1

    def body(s_ref, a_ref, w_ref, q_ref):
        x = jnp.concatenate(
            [s_ref[...].astype(jnp.bfloat16), a_ref[...].astype(jnp.bfloat16)],
            axis=1)
        w = w_ref[:row_bh, :].astype(jnp.bfloat16)
        h = jnp.dot(x, w, preferred_element_type=jnp.float32)
        h = jnp.maximum(h + w_ref[row_bh:row_bh + 1, :], 0.0)
        q = jnp.sum(h * w_ref[row_wo:row_wo + 1, :], axis=-1, keepdims=True)
        q_ref[...] = q + w_ref[row_bo:row_bo + 1, 0:1]

    return body


def kernel(s, a, w_all):
    B, s_dim = s.shape
    a_dim = a.shape[1]
    k_rows, hidden = w_all.shape

    tb = min(_TB, B) if B % min(_TB, B) == 0 else _TB
    pad = (-B) % tb
    if pad:
        s = jnp.pad(s, ((0, pad), (0, 0)))
        a = jnp.pad(a, ((0, pad), (0, 0)))
    bp = B + pad

    q = pl.pallas_call(
        _make_body(s_dim, a_dim),
        out_shape=jax.ShapeDtypeStruct((bp, 1), jnp.float32),
        grid=(bp // tb,),
        in_specs=[
            pl.BlockSpec((tb, s_dim), lambda i: (i, 0)),
            pl.BlockSpec((tb, a_dim), lambda i: (i, 0)),
            pl.BlockSpec((k_rows, hidden), lambda i: (0, 0)),
        ],
        out_specs=pl.BlockSpec((tb, 1), lambda i: (i, 0)),
        compiler_params=pltpu.CompilerParams(
            dimension_semantics=("parallel",),
            vmem_limit_bytes=64 << 20,
        ),
    )(s, a, w_all)
    return q[:B]


# speedup vs baseline: 2.2868x; 1.0484x over previous
import jax
import jax.numpy as jnp
from jax.experimental import pallas as pl
from jax.experimental.pallas import tpu as pltpu

_TB = 8192


def _make_body(s_dim, a_dim):
    row_bh = s_dim + a_dim
    row_wo = row_bh + 1
    row_bo = row_wo + 1

    def body(s_ref, a_ref, w_ref, q_ref):
        x = jnp.concatenate(
            [s_ref[...].astype(jnp.bfloat16), a_ref[...].astype(jnp.bfloat16)],
            axis=1)
        w = w_ref[:row_bh, :].astype(jnp.bfloat16)
        h = jnp.dot(x, w, preferred_element_type=jnp.float32)
        h = jnp.maximum(h + w_ref[row_bh:row_bh + 1, :], 0.0)
        q = jnp.sum(h * w_ref[row_wo:row_wo + 1, :], axis=-1, keepdims=True)
        q_ref[...] = q + w_ref[row_bo:row_bo + 1, 0:1]

    return body


def kernel(s, a, w_all):
    B, s_dim = s.shape
    a_dim = a.shape[1]
    k_rows, hidden = w_all.shape

    tb = min(_TB, B) if B % min(_TB, B) == 0 else _TB
    pad = (-B) % tb
    if pad:
        s = jnp.pad(s, ((0, pad), (0, 0)))
        a = jnp.pad(a, ((0, pad), (0, 0)))
    bp = B + pad

    q = pl.pallas_call(
        _make_body(s_dim, a_dim),
        out_shape=jax.ShapeDtypeStruct((bp, 1), jnp.float32),
        grid=(bp // tb,),
        in_specs=[
            pl.BlockSpec((tb, s_dim), lambda i: (i, 0)),
            pl.BlockSpec((tb, a_dim), lambda i: (i, 0)),
            pl.BlockSpec((k_rows, hidden), lambda i: (0, 0)),
        ],
        out_specs=pl.BlockSpec((tb, 1), lambda i: (i, 0)),
        compiler_params=pltpu.CompilerParams(
            dimension_semantics=("parallel",),
            vmem_limit_bytes=64 << 20,
        ),
    )(s, a, w_all)
    return q[:B]
